# baseline (speedup 1.0000x reference)
import functools

import jax
import jax.numpy as jnp
from jax import lax
from jax.experimental import pallas as pl
from jax.experimental.pallas import tpu as pltpu
from jax.experimental.pallas import tpu_sc as plsc

_N = 8192
_K = 8192
_C = 256
_RB = 256
_KB = 2048
_SB = 256
_RSTEPS = _N // _RB
_KSTEPS = _K // _KB

_NW = 32
_BPW = _N // _NW
_JCH = _BPW // 128


def _vq_body(x_ref, cb_ref, x2_ref, c2_ref, idx_ref, loss_ref,
             colf_ref, accv_ref, acci_ref, acc_ref):
    i = pl.program_id(0)
    j = pl.program_id(1)
    x = x_ref[...]
    x2 = x2_ref[...]

    @pl.when(jnp.logical_and(i == 0, j == 0))
    def _():
        colf_ref[...] = lax.broadcasted_iota(
            jnp.int32, (_RB, _KB), 1).astype(jnp.float32)

    run_m = None
    run_f = None
    for s in range(_KB // _SB):
        cbs = cb_ref[pl.ds(s * _SB, _SB), :]
        xc = lax.dot_general(x, cbs, (((1,), (1,)), ((), ())),
                             preferred_element_type=jnp.float32,
                             precision=lax.Precision.DEFAULT)
        c2 = c2_ref[:, pl.ds(s * _SB, _SB)]
        dist = (x2 - 2.0 * xc) + c2
        m_s = jnp.min(dist, axis=1, keepdims=True)
        f_s = jnp.min(
            jnp.where(dist == m_s, colf_ref[:, pl.ds(s * _SB, _SB)],
                      float(_K)), axis=1, keepdims=True)
        if s == 0:
            run_m, run_f = m_s, f_s
        else:
            b = m_s < run_m
            run_f = jnp.where(b, f_s, run_f)
            run_m = jnp.where(b, m_s, run_m)
    vmin = run_m
    pidx = run_f.astype(jnp.int32) + j * _KB

    @pl.when(j == 0)
    def _():
        accv_ref[...] = vmin
        acci_ref[...] = pidx

    @pl.when(j > 0)
    def _():
        thr = accv_ref[...].astype(jnp.bfloat16).astype(jnp.float32)
        take = vmin < thr
        accv_ref[...] = jnp.where(take, vmin, accv_ref[...])
        acci_ref[...] = jnp.where(take, pidx, acci_ref[...])

    @pl.when(jnp.logical_and(i == 0, j == 0))
    def _():
        acc_ref[...] = jnp.zeros((1, 1), jnp.float32)

    @pl.when(j == _KSTEPS - 1)
    def _():
        idx_ref[...] = acci_ref[...]
        acc_ref[...] = acc_ref[...] + jnp.sum(accv_ref[...])

    @pl.when(jnp.logical_and(i == _RSTEPS - 1, j == _KSTEPS - 1))
    def _():
        loss_ref[...] = acc_ref[...] * (1.0 / (_N * _C))


def _vq_tc(x_flat, codebook, x2, c2):
    return pl.pallas_call(
        _vq_body,
        grid=(_RSTEPS, _KSTEPS),
        in_specs=[
            pl.BlockSpec((_RB, _C), lambda i, j: (i, 0)),
            pl.BlockSpec((_KB, _C), lambda i, j: (j, 0)),
            pl.BlockSpec((_RB, 1), lambda i, j: (i, 0)),
            pl.BlockSpec((1, _KB), lambda i, j: (0, j)),
        ],
        out_specs=[
            pl.BlockSpec((_RB, 1), lambda i, j: (i, 0)),
            pl.BlockSpec((1, 1), lambda i, j: (0, 0)),
        ],
        out_shape=[
            jax.ShapeDtypeStruct((_N, 1), jnp.int32),
            jax.ShapeDtypeStruct((1, 1), jnp.float32),
        ],
        scratch_shapes=[
            pltpu.VMEM((_RB, _KB), jnp.float32),
            pltpu.VMEM((_RB, 1), jnp.float32),
            pltpu.VMEM((_RB, 1), jnp.int32),
            pltpu.VMEM((1, 1), jnp.float32),
        ],
        compiler_params=pltpu.CompilerParams(
            dimension_semantics=("arbitrary", "arbitrary"),
        ),
    )(x_flat, codebook, x2, c2)


def _sc_gather(codebook, idx3):
    mesh = plsc.VectorSubcoreMesh(core_axis_name="c", subcore_axis_name="s")

    @functools.partial(
        pl.kernel,
        out_type=jax.ShapeDtypeStruct((_N, _C), jnp.float32),
        mesh=mesh,
        scratch_types=[
            pltpu.VMEM((_JCH, 128), jnp.int32),
            pltpu.VMEM((128, _C), jnp.float32),
            pltpu.SemaphoreType.DMA,
        ],
    )
    def gk(cb_hbm, idx_hbm, out_hbm, idx_v, rows_v, sem):
        wid = lax.axis_index("s") * 2 + lax.axis_index("c")
        base = wid * _BPW
        pltpu.sync_copy(idx_hbm.at[wid], idx_v)
        for j in range(_JCH):
            pltpu.async_copy(cb_hbm.at[idx_v.at[j]], rows_v, sem).wait()
            pltpu.sync_copy(rows_v, out_hbm.at[pl.ds(base + j * 128, 128)])

    return gk(codebook, idx3)


def kernel(x, codebook):
    B, T, C = x.shape
    x_flat = x.reshape(-1, C)
    x2 = jnp.sum(x_flat * x_flat, axis=1, keepdims=True)
    c2 = jnp.sum(codebook * codebook, axis=1)
    idx2, loss = _vq_tc(x_flat, codebook, x2, c2.reshape(1, _K))
    idx_flat = idx2[:, 0]
    qe_flat = _sc_gather(codebook, idx_flat.reshape(_NW, _JCH, 128))
    qe = qe_flat.reshape(B, T, C)
    return (qe, loss[0, 0], idx_flat.reshape(B, T))


# speedup vs baseline: 1.2060x; 1.2060x over previous
import functools

import jax
import jax.numpy as jnp
from jax import lax
from jax.experimental import pallas as pl
from jax.experimental.pallas import tpu as pltpu
from jax.experimental.pallas import tpu_sc as plsc

_N = 8192
_K = 8192
_C = 256
_RB = 256
_KB = 2048
_SB = 512
_RSTEPS = _N // _RB
_KSTEPS = _K // _KB

_NW = 32
_BPW = _N // _NW
_JCH = _BPW // 128


def _vq_body(x_ref, cb_ref, x2_ref, c2_ref, idx_ref, loss_ref,
             colf_ref, accv_ref, acci_ref, acc_ref):
    j = pl.program_id(0)
    i = pl.program_id(1)
    x = x_ref[...]
    x2 = x2_ref[...]

    @pl.when(jnp.logical_and(i == 0, j == 0))
    def _():
        colf_ref[...] = lax.broadcasted_iota(
            jnp.int32, (_RB, _KB), 1).astype(jnp.float32)

    run_m = None
    run_f = None
    for s in range(_KB // _SB):
        cbs = cb_ref[pl.ds(s * _SB, _SB), :]
        xc = lax.dot_general(x, cbs, (((1,), (1,)), ((), ())),
                             preferred_element_type=jnp.float32,
                             precision=lax.Precision.DEFAULT)
        c2 = c2_ref[:, pl.ds(s * _SB, _SB)]
        dist = (x2 - 2.0 * xc) + c2
        m_s = jnp.min(dist, axis=1, keepdims=True)
        f_s = jnp.min(
            jnp.where(dist == m_s, colf_ref[:, pl.ds(s * _SB, _SB)],
                      float(_K)), axis=1, keepdims=True)
        if s == 0:
            run_m, run_f = m_s, f_s
        else:
            b = m_s < run_m
            run_f = jnp.where(b, f_s, run_f)
            run_m = jnp.where(b, m_s, run_m)
    vmin = run_m
    pidx = run_f.astype(jnp.int32) + j * _KB

    rs = pl.ds(i * _RB, _RB)

    @pl.when(j == 0)
    def _():
        accv_ref[rs, :] = vmin
        acci_ref[rs, :] = pidx

    @pl.when(j > 0)
    def _():
        prevv = accv_ref[rs, :]
        thr = prevv.astype(jnp.bfloat16).astype(jnp.float32)
        take = vmin < thr
        accv_ref[rs, :] = jnp.where(take, vmin, prevv)
        acci_ref[rs, :] = jnp.where(take, pidx, acci_ref[rs, :])

    @pl.when(jnp.logical_and(i == 0, j == 0))
    def _():
        acc_ref[...] = jnp.zeros((1, 1), jnp.float32)

    @pl.when(j == _KSTEPS - 1)
    def _():
        idx_ref[...] = acci_ref[rs, :]
        acc_ref[...] = acc_ref[...] + jnp.sum(accv_ref[rs, :])

    @pl.when(jnp.logical_and(i == _RSTEPS - 1, j == _KSTEPS - 1))
    def _():
        loss_ref[...] = acc_ref[...] * (1.0 / (_N * _C))


def _vq_tc(x_flat, codebook, x2, c2):
    return pl.pallas_call(
        _vq_body,
        grid=(_KSTEPS, _RSTEPS),
        in_specs=[
            pl.BlockSpec((_RB, _C), lambda j, i: (i, 0)),
            pl.BlockSpec((_KB, _C), lambda j, i: (j, 0)),
            pl.BlockSpec((_RB, 1), lambda j, i: (i, 0)),
            pl.BlockSpec((1, _KB), lambda j, i: (0, j)),
        ],
        out_specs=[
            pl.BlockSpec((_RB, 1), lambda j, i: (i, 0)),
            pl.BlockSpec((1, 1), lambda j, i: (0, 0)),
        ],
        out_shape=[
            jax.ShapeDtypeStruct((_N, 1), jnp.int32),
            jax.ShapeDtypeStruct((1, 1), jnp.float32),
        ],
        scratch_shapes=[
            pltpu.VMEM((_RB, _KB), jnp.float32),
            pltpu.VMEM((_N, 1), jnp.float32),
            pltpu.VMEM((_N, 1), jnp.int32),
            pltpu.VMEM((1, 1), jnp.float32),
        ],
        compiler_params=pltpu.CompilerParams(
            dimension_semantics=("arbitrary", "arbitrary"),
        ),
    )(x_flat, codebook, x2, c2)


def _sc_gather(codebook, idx3):
    mesh = plsc.VectorSubcoreMesh(core_axis_name="c", subcore_axis_name="s")

    @functools.partial(
        pl.kernel,
        out_type=jax.ShapeDtypeStruct((_N, _C), jnp.float32),
        mesh=mesh,
        scratch_types=[
            pltpu.VMEM((_JCH, 128), jnp.int32),
            pltpu.VMEM((128, _C), jnp.float32),
            pltpu.SemaphoreType.DMA,
        ],
    )
    def gk(cb_hbm, idx_hbm, out_hbm, idx_v, rows_v, sem):
        wid = lax.axis_index("s") * 2 + lax.axis_index("c")
        base = wid * _BPW
        pltpu.sync_copy(idx_hbm.at[wid], idx_v)
        for j in range(_JCH):
            pltpu.async_copy(cb_hbm.at[idx_v.at[j]], rows_v, sem).wait()
            pltpu.sync_copy(rows_v, out_hbm.at[pl.ds(base + j * 128, 128)])

    return gk(codebook, idx3)


def kernel(x, codebook):
    B, T, C = x.shape
    x_flat = x.reshape(-1, C)
    x2 = jnp.sum(x_flat * x_flat, axis=1, keepdims=True)
    c2 = jnp.sum(codebook * codebook, axis=1)
    idx2, loss = _vq_tc(x_flat, codebook, x2, c2.reshape(1, _K))
    idx_flat = idx2[:, 0]
    qe_flat = _sc_gather(codebook, idx_flat.reshape(_NW, _JCH, 128))
    qe = qe_flat.reshape(B, T, C)
    return (qe, loss[0, 0], idx_flat.reshape(B, T))
